# baseline (speedup 1.0000x reference)
import functools

import jax
import jax.numpy as jnp
from jax import lax
from jax.experimental import pallas as pl
from jax.experimental.pallas import tpu as pltpu
from jax.experimental.pallas import tpu_sc as plsc

DIM = 64
NCH = 8
NDG = DIM // 16
CB = 64
_IN_BOUNDS = lax.GatherScatterMode.PROMISE_IN_BOUNDS


def _take(vec, idx):
    return vec.at[idx].get(mode=_IN_BOUNDS)


@functools.lru_cache(maxsize=None)
def _make(B: int):
    info = plsc.get_sparse_core_info()
    NC, NS = info.num_cores, info.num_subcores
    NW = NC * NS
    BPW = B // NW
    NCHUNK = BPW // CB
    NFLAT = BPW * NCH
    NRSUB = CB * NCH * NDG // 128
    NWSUB = CB * NCH // 128
    scale = float(DIM ** 0.5 * NCH ** -0.5)

    mesh = plsc.VectorSubcoreMesh(core_axis_name="c", subcore_axis_name="s")

    @functools.partial(
        pl.kernel,
        mesh=mesh,
        out_type=jax.ShapeDtypeStruct((B, DIM), jnp.float32),
        compiler_params=pltpu.CompilerParams(use_tc_tiling_on_sc=False),
        scratch_types=[
            pltpu.VMEM((BPW,), jnp.int32),
            pltpu.VMEM((BPW,), jnp.int32),
            pltpu.VMEM((BPW, 16), jnp.int32),
            pltpu.VMEM((BPW, 16), jnp.int32),
            pltpu.VMEM((NFLAT * NDG,), jnp.int32),
            pltpu.VMEM((NFLAT,), jnp.int32),
            pltpu.VMEM((NFLAT,), jnp.int32),
            pltpu.VMEM((NWSUB, 128, 16), jnp.float32),
            pltpu.VMEM((NRSUB, 128, 16), jnp.float32),
            pltpu.VMEM((CB, DIM), jnp.float32),
            pltpu.SemaphoreType.DMA,
            pltpu.SemaphoreType.DMA,
        ],
    )
    def k(x_hbm, wt_hbm, h0p_hbm, h1p_hbm, out_hbm,
          x_v, xp_v, h0p_v, h1p_v, ir_v, i1r_v, i1s_v, w16_v, rows_v, o_v,
          sem0, sem1):
        wid = lax.axis_index("s") * NC + lax.axis_index("c")
        base = wid * BPW

        pltpu.sync_copy(x_hbm.at[pl.ds(base, BPW)], x_v)

        def halve(i, carry):
            xp_v[pl.ds(i * 16, 16)] = lax.shift_right_logical(
                x_v[pl.ds(i * 16, 16)], 1)
            return carry

        lax.fori_loop(0, BPW // 16, halve, 0)

        descs = []
        for j in range(BPW // 128):
            off = xp_v.at[pl.ds(j * 128, 128)]
            descs.append(pltpu.async_copy(
                h0p_hbm.at[off], h0p_v.at[pl.ds(j * 128, 128)], sem0))
            descs.append(pltpu.async_copy(
                h1p_hbm.at[off], h1p_v.at[pl.ds(j * 128, 128)], sem1))
        for d in descs:
            d.wait()

        lane = lax.iota(jnp.int32, 16)
        cvec = lax.bitwise_and(lane, 7)
        lo = lane < 8
        qsel = [lax.shift_right_logical(lane, 2) + 4 * q for q in range(4)]
        qadd = lax.bitwise_and(lane, 3)

        def flatten(g, carry):
            xg = x_v[pl.ds(g * 16, 16)]
            par8 = lax.bitwise_and(xg, 1) * 8
            for kk in range(8):
                b0 = g * 16 + 2 * kk
                r0h0 = h0p_v[b0]
                r1h0 = h0p_v[b0 + 1]
                r0h1 = h1p_v[b0]
                r1h1 = h1p_v[b0 + 1]
                i0 = cvec + par8[2 * kk]
                i1 = cvec + par8[2 * kk + 1]
                v0 = jnp.where(lo, _take(r0h0, i0), _take(r1h0, i1))
                v1 = jnp.where(lo, _take(r0h1, i0), _take(r1h1, i1))
                fb = g * 128 + kk * 16
                i1r_v[pl.ds(fb, 16)] = lax.shift_right_logical(v1, 4)
                i1s_v[pl.ds(fb, 16)] = lax.bitwise_and(v1, 15)
                v4 = v0 * 4
                for q in range(4):
                    ir_v[pl.ds(fb * 4 + q * 16, 16)] = _take(v4, qsel[q]) + qadd
            return carry

        lax.fori_loop(0, BPW // 16, flatten, 0)

        for chunk in range(NCHUNK):
            descs2 = [
                pltpu.async_copy(
                    wt_hbm.at[ir_v.at[pl.ds((chunk * NRSUB + j) * 128, 128)]],
                    rows_v.at[j], sem0)
                for j in range(NRSUB)
            ] + [
                pltpu.async_copy(
                    wt_hbm.at[i1r_v.at[pl.ds((chunk * NWSUB + j) * 128, 128)]],
                    w16_v.at[j], sem1)
                for j in range(NWSUB)
            ]
            for d in descs2:
                d.wait()

            def body(p, carry):
                sv = i1s_v[pl.ds((chunk * CB // 2 + p) * 16, 16)]
                for e in range(2):
                    b = p * 2 + e
                    wsub = lax.shift_right_logical(b, 4)
                    wrb = lax.bitwise_and(b, 15) * NCH
                    rsub = lax.shift_right_logical(b, 2)
                    rrb = lax.bitwise_and(b, 3) * (NCH * NDG)
                    acc = [None] * NDG
                    for c in range(NCH):
                        wsplat = _take(w16_v[wsub, wrb + c],
                                       jnp.broadcast_to(sv[e * NCH + c], (16,)))
                        for dg in range(NDG):
                            term = wsplat * rows_v[rsub, rrb + c * NDG + dg, :]
                            acc[dg] = term if c == 0 else acc[dg] + term
                    for dg in range(NDG):
                        o_v[b, pl.ds(dg * 16, 16)] = acc[dg] * scale
                return carry

            lax.fori_loop(0, CB // 2, body, 0)
            pltpu.sync_copy(o_v, out_hbm.at[pl.ds(base + chunk * CB, CB)])

    return k


def kernel(x, table, h0, h1):
    B = x.shape[0]
    V = h0.shape[0]
    wt = table.reshape(table.shape[0] * NDG, 16)
    h0p = h0.astype(jnp.int32).reshape(V // 2, 2 * NCH)
    h1p = h1.astype(jnp.int32).reshape(V // 2, 2 * NCH)
    return _make(B)(x.astype(jnp.int32), wt, h0p, h1p)


# speedup vs baseline: 2.1803x; 2.1803x over previous
import functools

import jax
import jax.numpy as jnp
from jax import lax
from jax.experimental import pallas as pl
from jax.experimental.pallas import tpu as pltpu
from jax.experimental.pallas import tpu_sc as plsc

DIM = 64
NCH = 8
NDG = DIM // 16
CB = 64
_IN_BOUNDS = lax.GatherScatterMode.PROMISE_IN_BOUNDS


def _take(vec, idx):
    return vec.at[idx].get(mode=_IN_BOUNDS)


@functools.lru_cache(maxsize=None)
def _make(B: int):
    info = plsc.get_sparse_core_info()
    NC, NS = info.num_cores, info.num_subcores
    NW = NC * NS
    BPW = B // NW
    NCHUNK = BPW // CB
    scale = float(DIM ** 0.5 * NCH ** -0.5)

    mesh = plsc.VectorSubcoreMesh(core_axis_name="c", subcore_axis_name="s")

    @functools.partial(
        pl.kernel,
        mesh=mesh,
        out_type=jax.ShapeDtypeStruct((B, DIM), jnp.float32),
        compiler_params=pltpu.CompilerParams(use_tc_tiling_on_sc=False),
        scratch_types=[
            pltpu.VMEM((BPW,), jnp.int32),
            pltpu.VMEM((NCH, BPW), jnp.int32),
            pltpu.VMEM((NCH, BPW), jnp.int32),
            pltpu.VMEM((NCH, BPW * NDG), jnp.int32),
            pltpu.VMEM((NCH, BPW), jnp.int32),
            pltpu.VMEM((NCH, BPW), jnp.int32),
            pltpu.VMEM((NCH, CB, 16), jnp.float32),
            pltpu.VMEM((NCH * 2, 128, 16), jnp.float32),
            pltpu.VMEM((CB, DIM), jnp.float32),
            pltpu.SemaphoreType.DMA,
            pltpu.SemaphoreType.DMA,
        ],
    )
    def k(x_hbm, wt_hbm,
          h0c0, h0c1, h0c2, h0c3, h0c4, h0c5, h0c6, h0c7,
          h1c0, h1c1, h1c2, h1c3, h1c4, h1c5, h1c6, h1c7,
          out_hbm,
          x_v, i0c_v, i1c_v, irc_v, i1r_v, i1s_v, w16_v, rows_v, o_v,
          sem0, sem1):
        h0c = [h0c0, h0c1, h0c2, h0c3, h0c4, h0c5, h0c6, h0c7]
        h1c = [h1c0, h1c1, h1c2, h1c3, h1c4, h1c5, h1c6, h1c7]
        wid = lax.axis_index("s") * NC + lax.axis_index("c")
        base = wid * BPW

        pltpu.sync_copy(x_hbm.at[pl.ds(base, BPW)], x_v)

        descs = []
        for j in range(BPW // 128):
            off = x_v.at[pl.ds(j * 128, 128)]
            for c in range(NCH):
                descs.append(pltpu.async_copy(
                    h0c[c].at[off], i0c_v.at[c, pl.ds(j * 128, 128)], sem0))
                descs.append(pltpu.async_copy(
                    h1c[c].at[off], i1c_v.at[c, pl.ds(j * 128, 128)], sem1))
        for d in descs:
            d.wait()

        lane = lax.iota(jnp.int32, 16)
        qsel = [lax.shift_right_logical(lane, 2) + 4 * q for q in range(4)]
        qadd = lax.bitwise_and(lane, 3)

        def expand(g, carry):
            c = lax.shift_right_logical(g, 5)
            gg = lax.bitwise_and(g, 31)
            v0 = i0c_v[c, pl.ds(gg * 16, 16)]
            v4 = v0 * 4
            for q in range(4):
                irc_v[c, pl.ds(gg * 64 + q * 16, 16)] = _take(v4, qsel[q]) + qadd
            v1 = i1c_v[c, pl.ds(gg * 16, 16)]
            i1r_v[c, pl.ds(gg * 16, 16)] = lax.shift_right_logical(v1, 4)
            i1s_v[c, pl.ds(gg * 16, 16)] = lax.bitwise_and(v1, 15)
            return carry

        lax.fori_loop(0, NCH * (BPW // 16), expand, 0)

        for chunk in range(NCHUNK):
            descs2 = []
            for c in range(NCH):
                for h in range(2):
                    descs2.append(pltpu.async_copy(
                        wt_hbm.at[irc_v.at[c, pl.ds(chunk * 256 + h * 128, 128)]],
                        rows_v.at[c * 2 + h], sem0))
                descs2.append(pltpu.async_copy(
                    wt_hbm.at[i1r_v.at[c, pl.ds(chunk * CB, CB)]],
                    w16_v.at[c], sem1))
            for d in descs2:
                d.wait()

            def body(b, carry):
                bsplat = jnp.broadcast_to(lax.bitwise_and(b, 15), (16,))
                bh = lax.shift_right_logical(b, 5)
                bl4 = lax.bitwise_and(b, 31) * 4
                bg16 = lax.shift_right_logical(b, 4) * 16
                acc = [None] * NDG
                for c in range(NCH):
                    sv = i1s_v[c, pl.ds(chunk * CB + bg16, 16)]
                    wrow = w16_v[c, b]
                    wsplat = _take(wrow, _take(sv, bsplat)) * scale
                    for q in range(NDG):
                        term = wsplat * rows_v[c * 2 + bh, bl4 + q, :]
                        acc[q] = term if c == 0 else acc[q] + term
                for q in range(NDG):
                    o_v[b, pl.ds(q * 16, 16)] = acc[q]
                return carry

            lax.fori_loop(0, CB, body, 0)
            pltpu.sync_copy(o_v, out_hbm.at[pl.ds(base + chunk * CB, CB)])

    return k


def kernel(x, table, h0, h1):
    B = x.shape[0]
    wt = table.reshape(table.shape[0] * NDG, 16)
    h0 = h0.astype(jnp.int32)
    h1 = h1.astype(jnp.int32)
    cols = [h0[:, c] for c in range(NCH)] + [h1[:, c] for c in range(NCH)]
    return _make(B)(x.astype(jnp.int32), wt, *cols)


# speedup vs baseline: 2.2340x; 1.0247x over previous
import functools

import jax
import jax.numpy as jnp
from jax import lax
from jax.experimental import pallas as pl
from jax.experimental.pallas import tpu as pltpu
from jax.experimental.pallas import tpu_sc as plsc

DIM = 64
NCH = 8
NDG = DIM // 16
CB = 64
_IN_BOUNDS = lax.GatherScatterMode.PROMISE_IN_BOUNDS


def _take(vec, idx):
    return vec.at[idx].get(mode=_IN_BOUNDS)


@functools.lru_cache(maxsize=None)
def _make(B: int):
    info = plsc.get_sparse_core_info()
    NC, NS = info.num_cores, info.num_subcores
    NW = NC * NS
    BPW = B // NW
    NCHUNK = BPW // CB
    scale = float(DIM ** 0.5 * NCH ** -0.5)

    mesh = plsc.VectorSubcoreMesh(core_axis_name="c", subcore_axis_name="s")

    @functools.partial(
        pl.kernel,
        mesh=mesh,
        out_type=jax.ShapeDtypeStruct((B, DIM), jnp.float32),
        compiler_params=pltpu.CompilerParams(use_tc_tiling_on_sc=False),
        scratch_types=[
            pltpu.VMEM((BPW,), jnp.int32),
            pltpu.VMEM((NCH, BPW), jnp.int32),
            pltpu.VMEM((NCH, BPW), jnp.int32),
            pltpu.VMEM((NCH, BPW), jnp.float32),
            pltpu.VMEM((2, NCH, CB, DIM), jnp.float32),
            pltpu.VMEM((CB, DIM), jnp.float32),
            pltpu.SemaphoreType.DMA,
            pltpu.SemaphoreType.DMA,
            pltpu.SemaphoreType.DMA,
        ],
    )
    def k(x_hbm, tab_hbm, flat_hbm,
          h0c0, h0c1, h0c2, h0c3, h0c4, h0c5, h0c6, h0c7,
          h1c0, h1c1, h1c2, h1c3, h1c4, h1c5, h1c6, h1c7,
          out_hbm,
          x_v, i0c_v, i1c_v, w_v, rows_v, o_v,
          semA, semB, semW):
        h0c = [h0c0, h0c1, h0c2, h0c3, h0c4, h0c5, h0c6, h0c7]
        h1c = [h1c0, h1c1, h1c2, h1c3, h1c4, h1c5, h1c6, h1c7]
        sem_par = [semA, semB]
        wid = lax.axis_index("s") * NC + lax.axis_index("c")
        base = wid * BPW

        pltpu.sync_copy(x_hbm.at[pl.ds(base, BPW)], x_v)

        descs = []
        for j in range(BPW // 128):
            off = x_v.at[pl.ds(j * 128, 128)]
            for c in range(NCH):
                descs.append(pltpu.async_copy(
                    h0c[c].at[off], i0c_v.at[c, pl.ds(j * 128, 128)], semW))
                descs.append(pltpu.async_copy(
                    h1c[c].at[off], i1c_v.at[c, pl.ds(j * 128, 128)], semW))
        for d in descs:
            d.wait()

        wdescs = []
        for c in range(NCH):
            for j in range(BPW // 128):
                wdescs.append(pltpu.async_copy(
                    flat_hbm.at[i1c_v.at[c, pl.ds(j * 128, 128)]],
                    w_v.at[c, pl.ds(j * 128, 128)], semW))

        def issue_rows(chunk):
            par = chunk % 2
            return [pltpu.async_copy(
                tab_hbm.at[i0c_v.at[c, pl.ds(chunk * CB, CB)]],
                rows_v.at[par, c], sem_par[par]) for c in range(NCH)]

        pending = issue_rows(0)
        for d in wdescs:
            d.wait()

        for chunk in range(NCHUNK):
            par = chunk % 2
            for d in pending:
                d.wait()
            if chunk + 1 < NCHUNK:
                pending = issue_rows(chunk + 1)

            def body(b, carry):
                bsplat = jnp.broadcast_to(lax.bitwise_and(b, 15), (16,))
                bg16 = lax.shift_right_logical(b, 4) * 16
                acc = [None] * NDG
                for c in range(NCH):
                    wrow = w_v[c, pl.ds(chunk * CB + bg16, 16)]
                    wsplat = _take(wrow, bsplat) * scale
                    for q in range(NDG):
                        term = wsplat * rows_v[par, c, b, pl.ds(q * 16, 16)]
                        acc[q] = term if c == 0 else acc[q] + term
                for q in range(NDG):
                    o_v[b, pl.ds(q * 16, 16)] = acc[q]
                return carry

            lax.fori_loop(0, CB, body, 0)
            pltpu.sync_copy(o_v, out_hbm.at[pl.ds(base + chunk * CB, CB)])

    return k


def kernel(x, table, h0, h1):
    B = x.shape[0]
    h0 = h0.astype(jnp.int32)
    h1 = h1.astype(jnp.int32)
    cols = [h0[:, c] for c in range(NCH)] + [h1[:, c] for c in range(NCH)]
    flat = table.reshape(-1)
    flat = flat.at[0].set(flat[0])
    return _make(B)(x.astype(jnp.int32), table, flat, *cols)
